# baseline (speedup 1.0000x reference)
import jax
import jax.numpy as jnp
from jax.experimental import pallas as pl


def _fusion_kernel(u_ref, e_ref, wq0_ref, wk0_ref, wv0_ref, wp0_ref, we0_ref,
                   wq1_ref, wk1_ref, wv1_ref, wp1_ref, o_ref):
    f32, bf16 = jnp.float32, jnp.bfloat16
    u = u_ref[0]
    e8 = e_ref[0]
    e = jnp.concatenate([e8, jnp.mean(e8, axis=0, keepdims=True)],
                        axis=0).astype(bf16)
    d = u.shape[-1]
    scale = jax.lax.rsqrt(f32(d))

    def mm(a, b):
        return jnp.dot(a, b, preferred_element_type=f32)

    def attend(q, k, v):
        lg = jax.lax.dot_general(q.astype(bf16), k.astype(bf16),
                                 (((1,), (1,)), ((), ())),
                                 preferred_element_type=f32) * scale
        lg = lg - jnp.max(lg, axis=-1, keepdims=True)
        w = jnp.exp(lg)
        a = w / jnp.sum(w, axis=-1, keepdims=True)
        return mm(a.astype(bf16), v.astype(bf16))

    inj0 = attend(mm(u, wq0_ref[...]), mm(e, wk0_ref[...]), mm(e, wv0_ref[...]))
    xm = mm(u, wp0_ref[...]) + inj0
    xmb = xm.astype(bf16)
    e1 = mm(e, we0_ref[...]).astype(bf16)
    inj1 = attend(mm(xmb, wq1_ref[...]), mm(e1, wk1_ref[...]), mm(e1, wv1_ref[...]))
    o_ref[0] = mm(xmb, wp1_ref[...]) + inj1


def kernel(patches, embs, locations, Wq0, Wk0, Wv0, Wp0, We0,
           Wq1, Wk1, Wv1, Wp1, We1):
    del locations, We1
    b, h, w, d0 = patches.shape
    p = h * w
    m0 = embs.shape[1]
    u = patches.reshape(b, p, d0).astype(jnp.bfloat16)
    ws = [w.astype(jnp.bfloat16) for w in (Wq0, Wk0, Wv0, Wp0, We0, Wq1, Wk1, Wv1, Wp1)]
    wspec = pl.BlockSpec((d0, d0), lambda i: (0, 0))
    return pl.pallas_call(
        _fusion_kernel,
        grid=(b,),
        in_specs=[pl.BlockSpec((1, p, d0), lambda i: (i, 0, 0)),
                  pl.BlockSpec((1, m0, d0), lambda i: (i, 0, 0))] + [wspec] * 9,
        out_specs=pl.BlockSpec((1, p, d0), lambda i: (i, 0, 0)),
        out_shape=jax.ShapeDtypeStruct((b, p, d0), jnp.float32),
    )(u, embs, *ws)


# speedup vs baseline: 1.4060x; 1.4060x over previous
import jax
import jax.numpy as jnp
from jax.experimental import pallas as pl


def _fusion_kernel(u_ref, e_ref, wq0_ref, wk0_ref, wv0_ref, wp0_ref, we0_ref,
                   wq1_ref, wk1_ref, wv1_ref, wp1_ref, o_ref):
    f32, bf16 = jnp.float32, jnp.bfloat16
    b, m0, d = e_ref.shape
    p = u_ref.shape[0] // b
    m = m0 + 1
    scale = jax.lax.rsqrt(f32(d))

    def mm(a, w):
        return jnp.dot(a, w, preferred_element_type=f32)

    ub = u_ref[...].astype(bf16)
    e_parts = []
    for i in range(b):
        e_i = e_ref[i]
        e_parts.append(e_i)
        e_parts.append(jnp.mean(e_i, axis=0, keepdims=True))
    e36 = jnp.concatenate(e_parts, axis=0).astype(bf16)

    def attend(q, k, v):
        kb, vb = k.astype(bf16), v.astype(bf16)
        qb = q.astype(bf16)
        outs = []
        for i in range(b):
            lg = jax.lax.dot_general(
                qb[i * p:(i + 1) * p], kb[i * m:(i + 1) * m],
                (((1,), (1,)), ((), ())), preferred_element_type=f32) * scale
            lg = lg - jnp.max(lg, axis=-1, keepdims=True)
            ex = jnp.exp(lg)
            a = (ex / jnp.sum(ex, axis=-1, keepdims=True)).astype(bf16)
            outs.append(mm(a, vb[i * m:(i + 1) * m]))
        return jnp.concatenate(outs, axis=0)

    inj0 = attend(mm(ub, wq0_ref[...].astype(bf16)),
                  mm(e36, wk0_ref[...].astype(bf16)),
                  mm(e36, wv0_ref[...].astype(bf16)))
    xm = mm(ub, wp0_ref[...].astype(bf16)) + inj0
    xmb = xm.astype(bf16)
    e1 = mm(e36, we0_ref[...].astype(bf16)).astype(bf16)
    inj1 = attend(mm(xmb, wq1_ref[...].astype(bf16)),
                  mm(e1, wk1_ref[...].astype(bf16)),
                  mm(e1, wv1_ref[...].astype(bf16)))
    o_ref[...] = mm(xmb, wp1_ref[...].astype(bf16)) + inj1


def kernel(patches, embs, locations, Wq0, Wk0, Wv0, Wp0, We0,
           Wq1, Wk1, Wv1, Wp1, We1):
    del locations, We1
    b, h, w, d0 = patches.shape
    p = h * w
    u = patches.reshape(b * p, d0)
    out = pl.pallas_call(
        _fusion_kernel,
        out_shape=jax.ShapeDtypeStruct((b * p, d0), jnp.float32),
    )(u, embs, Wq0, Wk0, Wv0, Wp0, We0, Wq1, Wk1, Wv1, Wp1)
    return out.reshape(b, p, d0)


# speedup vs baseline: 1.4432x; 1.0265x over previous
import jax
import jax.numpy as jnp
from jax.experimental import pallas as pl
from jax.experimental.pallas import tpu as pltpu


def _fusion_kernel(e_ref, u_hbm, wk0_hbm, wv0_hbm, wq0_hbm, wp0_hbm, we0_hbm,
                   wk1_hbm, wv1_hbm, wq1_hbm, wp1_hbm, o_ref,
                   u_s, w_s, injf_s, xmb_s, sems):
    f32, bf16 = jnp.float32, jnp.bfloat16
    b, m0, d = e_ref.shape
    p = u_hbm.shape[0] // b
    m = m0 + 1
    scale = jax.lax.rsqrt(f32(d))

    srcs = [wk0_hbm, wv0_hbm, u_hbm, wq0_hbm, wp0_hbm, we0_hbm,
            wk1_hbm, wv1_hbm, wq1_hbm, wp1_hbm]
    copies = []
    wslot_idx = 0
    for i, src in enumerate(srcs):
        if src is u_hbm:
            dst = u_s
        else:
            dst = w_s.at[wslot_idx]
            wslot_idx += 1
        cp = pltpu.make_async_copy(src, dst, sems.at[i])
        cp.start()
        copies.append(cp)
    (cp_k0, cp_v0, cp_u, cp_q0, cp_p0, cp_e0,
     cp_k1, cp_v1, cp_q1, cp_p1) = copies

    def mm(a, w):
        return jnp.dot(a, w, preferred_element_type=f32)

    def wslot(i):
        return w_s[i].astype(bf16)

    e_parts = []
    for i in range(b):
        e_i = e_ref[i]
        e_parts.append(e_i)
        e_parts.append(jnp.mean(e_i, axis=0, keepdims=True))
    e36 = jnp.concatenate(e_parts, axis=0).astype(bf16)

    def attend(q, k, v):
        qb = q.astype(bf16)
        outs = []
        for i in range(b):
            lg = jax.lax.dot_general(
                qb[i * p:(i + 1) * p], k[i * m:(i + 1) * m],
                (((1,), (1,)), ((), ())), preferred_element_type=f32) * scale
            lg = lg - jnp.max(lg, axis=-1, keepdims=True)
            ex = jnp.exp(lg)
            a = (ex / jnp.sum(ex, axis=-1, keepdims=True)).astype(bf16)
            outs.append(mm(a, v[i * m:(i + 1) * m]))
        return jnp.concatenate(outs, axis=0)

    cp_k0.wait()
    k0 = mm(e36, wslot(0)).astype(bf16)
    cp_v0.wait()
    v0 = mm(e36, wslot(1)).astype(bf16)
    cp_u.wait()
    ub = u_s[...].astype(bf16)
    cp_q0.wait()
    injf_s[...] = attend(mm(ub, wslot(2)), k0, v0)
    cp_p0.wait()
    xmb_s[...] = (mm(ub, wslot(3)) + injf_s[...]).astype(bf16)
    cp_e0.wait()
    e1 = mm(e36, wslot(4)).astype(bf16)
    cp_k1.wait()
    k1 = mm(e1, wslot(5)).astype(bf16)
    cp_v1.wait()
    v1 = mm(e1, wslot(6)).astype(bf16)
    cp_q1.wait()
    xmb = xmb_s[...]
    inj1 = attend(mm(xmb, wslot(7)), k1, v1)
    cp_p1.wait()
    o_ref[...] = mm(xmb, wslot(8)) + inj1


def kernel(patches, embs, locations, Wq0, Wk0, Wv0, Wp0, We0,
           Wq1, Wk1, Wv1, Wp1, We1):
    del locations, We1
    b, h, w, d0 = patches.shape
    p = h * w
    u = patches.reshape(b * p, d0)
    hbm = pl.BlockSpec(memory_space=pltpu.MemorySpace.HBM)
    out = pl.pallas_call(
        _fusion_kernel,
        in_specs=[pl.BlockSpec(memory_space=pltpu.MemorySpace.VMEM)] + [hbm] * 10,
        out_specs=pl.BlockSpec(memory_space=pltpu.MemorySpace.VMEM),
        out_shape=jax.ShapeDtypeStruct((b * p, d0), jnp.float32),
        scratch_shapes=[
            pltpu.VMEM((b * p, d0), jnp.float32),
            pltpu.VMEM((9, d0, d0), jnp.float32),
            pltpu.VMEM((b * p, d0), jnp.float32),
            pltpu.VMEM((b * p, d0), jnp.bfloat16),
            pltpu.SemaphoreType.DMA((10,)),
        ],
    )(embs, u, Wk0, Wv0, Wq0, Wp0, We0, Wk1, Wv1, Wq1, Wp1)
    return out.reshape(b, p, d0)


# speedup vs baseline: 1.5543x; 1.0770x over previous
import jax
import jax.numpy as jnp
from jax.experimental import pallas as pl
from jax.experimental.pallas import tpu as pltpu


def _fusion_kernel(e_ref, u_hbm, wk0_hbm, wv0_hbm, we0_hbm, wq0_hbm, wp0_hbm,
                   wk1_hbm, wv1_hbm, wq1_hbm, wp1_hbm, o_ref,
                   u_s, wa_s, wc_s, wq0_s, wp0_s, wq1_s, wp1_s, sems):
    f32, bf16 = jnp.float32, jnp.bfloat16
    b, m0, d = e_ref.shape
    p = u_hbm.shape[0] // b
    m = m0 + 1
    scale = jax.lax.rsqrt(f32(d))

    plan = [
        (wk0_hbm, wa_s.at[:, 0 * d:1 * d]),
        (wv0_hbm, wa_s.at[:, 1 * d:2 * d]),
        (we0_hbm, wa_s.at[:, 2 * d:3 * d]),
        (u_hbm, u_s),
        (wq0_hbm, wq0_s),
        (wp0_hbm, wp0_s),
        (wk1_hbm, wc_s.at[:, 0 * d:1 * d]),
        (wv1_hbm, wc_s.at[:, 1 * d:2 * d]),
        (wq1_hbm, wq1_s),
        (wp1_hbm, wp1_s),
    ]
    copies = []
    for i, (src, dst) in enumerate(plan):
        cp = pltpu.make_async_copy(src, dst, sems.at[i])
        cp.start()
        copies.append(cp)
    (cp_k0, cp_v0, cp_e0, cp_u, cp_q0, cp_p0,
     cp_k1, cp_v1, cp_q1, cp_p1) = copies

    def mm(a, w):
        return jnp.dot(a, w, preferred_element_type=f32)

    e_parts = []
    for i in range(b):
        e_i = e_ref[i]
        e_parts.append(e_i)
        e_parts.append(jnp.mean(e_i, axis=0, keepdims=True))
    e36 = jnp.concatenate(e_parts, axis=0).astype(bf16)

    ri = jax.lax.broadcasted_iota(jnp.int32, (b * p, b * m), 0)
    ci = jax.lax.broadcasted_iota(jnp.int32, (b * p, b * m), 1)
    rb = sum((ri >= k * p).astype(jnp.int32) for k in range(1, b))
    cb = sum((ci >= k * m).astype(jnp.int32) for k in range(1, b))
    bias = jnp.where(rb == cb, f32(0), f32(-1e30))

    def attend(q, k36, v36):
        lg = jax.lax.dot_general(q.astype(bf16), k36,
                                 (((1,), (1,)), ((), ())),
                                 preferred_element_type=f32) * scale + bias
        lg = lg - jnp.max(lg, axis=-1, keepdims=True)
        ex = jnp.exp(lg)
        a = (ex / jnp.sum(ex, axis=-1, keepdims=True)).astype(bf16)
        return mm(a, v36)

    cp_k0.wait(); cp_v0.wait(); cp_e0.wait()
    kve = mm(e36, wa_s[...].astype(bf16))
    k0 = kve[:, 0 * d:1 * d].astype(bf16)
    v0 = kve[:, 1 * d:2 * d].astype(bf16)
    e1 = kve[:, 2 * d:3 * d].astype(bf16)
    cp_u.wait()
    ub = u_s[...].astype(bf16)
    cp_q0.wait()
    inj0 = attend(mm(ub, wq0_s[...].astype(bf16)), k0, v0)
    cp_p0.wait()
    xmb = (mm(ub, wp0_s[...].astype(bf16)) + inj0).astype(bf16)
    cp_k1.wait(); cp_v1.wait()
    kv1 = mm(e1, wc_s[...].astype(bf16))
    k1 = kv1[:, 0 * d:1 * d].astype(bf16)
    v1 = kv1[:, 1 * d:2 * d].astype(bf16)
    cp_q1.wait()
    inj1 = attend(mm(xmb, wq1_s[...].astype(bf16)), k1, v1)
    cp_p1.wait()
    o_ref[...] = mm(xmb, wp1_s[...].astype(bf16)) + inj1


def kernel(patches, embs, locations, Wq0, Wk0, Wv0, Wp0, We0,
           Wq1, Wk1, Wv1, Wp1, We1):
    del locations, We1
    b, h, w, d0 = patches.shape
    p = h * w
    u = patches.reshape(b * p, d0)
    hbm = pl.BlockSpec(memory_space=pltpu.MemorySpace.HBM)
    f32 = jnp.float32
    out = pl.pallas_call(
        _fusion_kernel,
        in_specs=[pl.BlockSpec(memory_space=pltpu.MemorySpace.VMEM)] + [hbm] * 10,
        out_specs=pl.BlockSpec(memory_space=pltpu.MemorySpace.VMEM),
        out_shape=jax.ShapeDtypeStruct((b * p, d0), f32),
        scratch_shapes=[
            pltpu.VMEM((b * p, d0), f32),
            pltpu.VMEM((d0, 3 * d0), f32),
            pltpu.VMEM((d0, 2 * d0), f32),
            pltpu.VMEM((d0, d0), f32),
            pltpu.VMEM((d0, d0), f32),
            pltpu.VMEM((d0, d0), f32),
            pltpu.VMEM((d0, d0), f32),
            pltpu.SemaphoreType.DMA((10,)),
        ],
    )(embs, u, Wk0, Wv0, We0, Wq0, Wp0, Wk1, Wv1, Wq1, Wp1)
    return out.reshape(b, p, d0)


# speedup vs baseline: 1.7910x; 1.1523x over previous
import jax
import jax.numpy as jnp
from jax.experimental import pallas as pl
from jax.experimental.pallas import tpu as pltpu


def _fusion_kernel(e_ref, u_hbm, wk0_hbm, wv0_hbm, we0_hbm, wq0_hbm, wp0_hbm,
                   wk1_hbm, wv1_hbm, wq1_hbm, wp1_hbm, o_ref,
                   u_s, wa_s, wc_s, wq0_s, wp0_s, wq1_s, wp1_s, sems):
    f32, bf16 = jnp.float32, jnp.bfloat16
    b, m0, d = e_ref.shape
    p = u_hbm.shape[0] // b
    m = m0 + 1
    scale = jax.lax.rsqrt(f32(d))

    plan = [
        (wk0_hbm, wa_s.at[:, 0 * d:1 * d]),
        (wv0_hbm, wa_s.at[:, 1 * d:2 * d]),
        (we0_hbm, wa_s.at[:, 2 * d:3 * d]),
        (wq0_hbm, wq0_s),
        (u_hbm, u_s),
        (wp0_hbm, wp0_s),
        (wk1_hbm, wc_s.at[:, 0 * d:1 * d]),
        (wv1_hbm, wc_s.at[:, 1 * d:2 * d]),
        (wq1_hbm, wq1_s),
        (wp1_hbm, wp1_s),
    ]
    copies = []
    for i, (src, dst) in enumerate(plan):
        cp = pltpu.make_async_copy(src, dst, sems.at[i])
        cp.start()
        copies.append(cp)
    (cp_k0, cp_v0, cp_e0, cp_q0, cp_u, cp_p0,
     cp_k1, cp_v1, cp_q1, cp_p1) = copies

    def mm(a, w):
        return jnp.dot(a, w, preferred_element_type=f32)

    e_parts = []
    for i in range(b):
        e_i = e_ref[i]
        e_parts.append(e_i)
        e_parts.append(jnp.mean(e_i, axis=0, keepdims=True))
    e36 = jnp.concatenate(e_parts, axis=0).astype(bf16)

    ri = jax.lax.broadcasted_iota(jnp.int32, (b * p, b * m), 0)
    ci = jax.lax.broadcasted_iota(jnp.int32, (b * p, b * m), 1)
    rb = sum((ri >= k * p).astype(jnp.int32) for k in range(1, b))
    cb = sum((ci >= k * m).astype(jnp.int32) for k in range(1, b))
    bias = jnp.where(rb == cb, f32(0), f32(-1e30))

    def dgT(a, c):
        return jax.lax.dot_general(a, c, (((1,), (1,)), ((), ())),
                                   preferred_element_type=f32)

    def attend(x, g36, v36):
        lg = dgT(x, g36) + bias
        lg = lg - jnp.max(lg, axis=-1, keepdims=True)
        ex = jnp.exp(lg)
        a = (ex / jnp.sum(ex, axis=-1, keepdims=True)).astype(bf16)
        return mm(a, v36)

    cp_k0.wait(); cp_v0.wait(); cp_e0.wait()
    kve = mm(e36, wa_s[...].astype(bf16))
    k0 = kve[:, 0 * d:1 * d].astype(bf16)
    v0 = kve[:, 1 * d:2 * d].astype(bf16)
    e1 = kve[:, 2 * d:3 * d].astype(bf16)
    cp_q0.wait()
    g0 = (dgT(k0, wq0_s[...].astype(bf16)) * scale).astype(bf16)
    cp_u.wait()
    ub = u_s[...].astype(bf16)
    inj0 = attend(ub, g0, v0)
    cp_p0.wait()
    xmb = (mm(ub, wp0_s[...].astype(bf16)) + inj0).astype(bf16)
    cp_k1.wait(); cp_v1.wait()
    kv1 = mm(e1, wc_s[...].astype(bf16))
    k1 = kv1[:, 0 * d:1 * d].astype(bf16)
    v1 = kv1[:, 1 * d:2 * d].astype(bf16)
    cp_q1.wait()
    g1 = (dgT(k1, wq1_s[...].astype(bf16)) * scale).astype(bf16)
    inj1 = attend(xmb, g1, v1)
    cp_p1.wait()
    o_ref[...] = mm(xmb, wp1_s[...].astype(bf16)) + inj1


def kernel(patches, embs, locations, Wq0, Wk0, Wv0, Wp0, We0,
           Wq1, Wk1, Wv1, Wp1, We1):
    del locations, We1
    b, h, w, d0 = patches.shape
    p = h * w
    u = patches.reshape(b * p, d0)
    hbm = pl.BlockSpec(memory_space=pltpu.MemorySpace.HBM)
    f32 = jnp.float32
    out = pl.pallas_call(
        _fusion_kernel,
        in_specs=[pl.BlockSpec(memory_space=pltpu.MemorySpace.VMEM)] + [hbm] * 10,
        out_specs=pl.BlockSpec(memory_space=pltpu.MemorySpace.VMEM),
        out_shape=jax.ShapeDtypeStruct((b * p, d0), f32),
        scratch_shapes=[
            pltpu.VMEM((b * p, d0), f32),
            pltpu.VMEM((d0, 3 * d0), f32),
            pltpu.VMEM((d0, 2 * d0), f32),
            pltpu.VMEM((d0, d0), f32),
            pltpu.VMEM((d0, d0), f32),
            pltpu.VMEM((d0, d0), f32),
            pltpu.VMEM((d0, d0), f32),
            pltpu.SemaphoreType.DMA((10,)),
        ],
    )(embs, u, Wk0, Wv0, We0, Wq0, Wp0, Wk1, Wv1, Wq1, Wp1)
    return out.reshape(b, p, d0)
